# baseline (device time: 44354 ns/iter reference)
import jax
import jax.numpy as jnp
from jax import lax
from jax.experimental import pallas as pl
from jax.experimental.pallas import tpu as pltpu

N_DEV = 4


def kernel(x):
    m_per, n = x.shape
    m_half = m_per // 2

    def body(x_ref, out_ref, g, sr, rr, sl, rl, csems):
        my = lax.axis_index("i")
        left = (my - 1) % N_DEV
        right = (my + 1) % N_DEV

        barrier_sem = pltpu.get_barrier_semaphore()
        for nbr in (left, right):
            pl.semaphore_signal(
                barrier_sem, inc=1,
                device_id=(nbr,), device_id_type=pl.DeviceIdType.MESH,
            )
        pl.semaphore_wait(barrier_sem, 2)

        g[pl.ds(my * m_per, m_per), :] = x_ref[:, :].astype(jnp.bfloat16)

        def copy(row_start, rows, dev, ssem, rsem):
            return pltpu.make_async_remote_copy(
                src_ref=g.at[pl.ds(row_start, rows), :],
                dst_ref=g.at[pl.ds(row_start, rows), :],
                send_sem=ssem, recv_sem=rsem,
                device_id=(dev,), device_id_type=pl.DeviceIdType.MESH,
            )

        def drain(row_start, rows, sem):
            c = pltpu.make_async_copy(
                g.at[pl.ds(row_start, rows), :],
                out_ref.at[pl.ds(row_start, rows), :],
                sem,
            )
            c.start()
            return c

        base = my * m_per
        r0a = copy(base, m_half, right, sr.at[0], rr.at[0])
        r0b = copy(base + m_half, m_half, right, sr.at[1], rr.at[1])
        l0a = copy(base + m_half, m_half, left, sl.at[0], rl.at[0])
        l0b = copy(base, m_half, left, sl.at[1], rl.at[1])
        r0a.start()
        r0b.start()
        l0a.start()
        l0b.start()
        c_own = drain(base, m_per, csems.at[0])

        r0a.wait_recv()
        r1 = copy(left * m_per, m_half, right, sr.at[2], rr.at[2])
        r1.start()

        l0a.wait_recv()
        l1 = copy(right * m_per + m_half, m_half, left, sl.at[2], rl.at[2])
        l1.start()

        r0b.wait_recv()
        c_left = drain(left * m_per, m_per, csems.at[1])
        l0b.wait_recv()
        c_right = drain(right * m_per, m_per, csems.at[2])

        r1.wait_recv()
        l1.wait_recv()
        c_diag = drain(((my + 2) % N_DEV) * m_per, m_per, csems.at[3])

        for rdma in (r0a, r0b, l0a, l0b, r1, l1):
            rdma.wait_send()
        for c in (c_own, c_left, c_right, c_diag):
            c.wait()

    return pl.pallas_call(
        body,
        out_shape=jax.ShapeDtypeStruct((N_DEV * m_per, n), jnp.bfloat16),
        in_specs=[pl.BlockSpec(memory_space=pltpu.VMEM)],
        out_specs=pl.BlockSpec(memory_space=pltpu.MemorySpace.HBM),
        scratch_shapes=[
            pltpu.VMEM((N_DEV * m_per, n), jnp.bfloat16),
            pltpu.SemaphoreType.DMA((3,)),
            pltpu.SemaphoreType.DMA((3,)),
            pltpu.SemaphoreType.DMA((3,)),
            pltpu.SemaphoreType.DMA((3,)),
            pltpu.SemaphoreType.DMA((4,)),
        ],
        compiler_params=pltpu.CompilerParams(collective_id=0),
    )(x)


# device time: 43027 ns/iter; 1.0308x vs baseline; 1.0308x over previous
import jax
import jax.numpy as jnp
from jax import lax
from jax.experimental import pallas as pl
from jax.experimental.pallas import tpu as pltpu

N_DEV = 4


def kernel(x):
    m_per, n = x.shape
    m_half = m_per // 2

    def body(x_ref, out_ref, xv, g, sr, rr, sl, rl, csems, isems):
        my = lax.axis_index("i")
        left = (my - 1) % N_DEV
        right = (my + 1) % N_DEV

        cin_a = pltpu.make_async_copy(
            x_ref.at[pl.ds(0, m_half), :], xv.at[pl.ds(0, m_half), :],
            isems.at[0])
        cin_b = pltpu.make_async_copy(
            x_ref.at[pl.ds(m_half, m_half), :], xv.at[pl.ds(m_half, m_half), :],
            isems.at[1])
        cin_a.start()
        cin_b.start()

        barrier_sem = pltpu.get_barrier_semaphore()
        for nbr in (left, right):
            pl.semaphore_signal(
                barrier_sem, inc=1,
                device_id=(nbr,), device_id_type=pl.DeviceIdType.MESH,
            )
        pl.semaphore_wait(barrier_sem, 2)

        def copy(row_start, rows, dev, ssem, rsem):
            return pltpu.make_async_remote_copy(
                src_ref=g.at[pl.ds(row_start, rows), :],
                dst_ref=g.at[pl.ds(row_start, rows), :],
                send_sem=ssem, recv_sem=rsem,
                device_id=(dev,), device_id_type=pl.DeviceIdType.MESH,
            )

        def drain(row_start, rows, sem):
            c = pltpu.make_async_copy(
                g.at[pl.ds(row_start, rows), :],
                out_ref.at[pl.ds(row_start, rows), :],
                sem,
            )
            c.start()
            return c

        base = my * m_per
        r0a = copy(base, m_half, right, sr.at[0], rr.at[0])
        r0b = copy(base + m_half, m_half, right, sr.at[1], rr.at[1])
        l0a = copy(base + m_half, m_half, left, sl.at[0], rl.at[0])
        l0b = copy(base, m_half, left, sl.at[1], rl.at[1])

        cin_a.wait()
        g[pl.ds(base, m_half), :] = xv[:m_half, :].astype(jnp.bfloat16)
        r0a.start()
        cin_b.wait()
        g[pl.ds(base + m_half, m_half), :] = xv[m_half:, :].astype(jnp.bfloat16)
        l0a.start()
        r0b.start()
        l0b.start()
        c_own = drain(base, m_per, csems.at[0])

        r0a.wait_recv()
        r1 = copy(left * m_per, m_half, right, sr.at[2], rr.at[2])
        r1.start()

        l0a.wait_recv()
        l1 = copy(right * m_per + m_half, m_half, left, sl.at[2], rl.at[2])
        l1.start()

        r0b.wait_recv()
        c_left = drain(left * m_per, m_per, csems.at[1])
        l0b.wait_recv()
        c_right = drain(right * m_per, m_per, csems.at[2])

        r1.wait_recv()
        l1.wait_recv()
        c_diag = drain(((my + 2) % N_DEV) * m_per, m_per, csems.at[3])

        for rdma in (r0a, r0b, l0a, l0b, r1, l1):
            rdma.wait_send()
        for c in (c_own, c_left, c_right, c_diag):
            c.wait()

    x_hbm = pltpu.with_memory_space_constraint(x, pltpu.MemorySpace.HBM)
    return pl.pallas_call(
        body,
        out_shape=jax.ShapeDtypeStruct((N_DEV * m_per, n), jnp.bfloat16),
        in_specs=[
            pl.BlockSpec(memory_space=pl.ANY),
        ],
        out_specs=pl.BlockSpec(memory_space=pl.ANY),
        scratch_shapes=[
            pltpu.VMEM((m_per, n), x.dtype),
            pltpu.VMEM((N_DEV * m_per, n), jnp.bfloat16),
            pltpu.SemaphoreType.DMA((3,)),
            pltpu.SemaphoreType.DMA((3,)),
            pltpu.SemaphoreType.DMA((3,)),
            pltpu.SemaphoreType.DMA((3,)),
            pltpu.SemaphoreType.DMA((4,)),
            pltpu.SemaphoreType.DMA((2,)),
        ],
        compiler_params=pltpu.CompilerParams(collective_id=0),
    )(x_hbm)


# device time: 42846 ns/iter; 1.0352x vs baseline; 1.0042x over previous
import jax
import jax.numpy as jnp
from jax import lax
from jax.experimental import pallas as pl
from jax.experimental.pallas import tpu as pltpu

N_DEV = 4


def kernel(x):
    m_per, n = x.shape
    m_half = m_per // 2

    def body(x_ref, out_ref, xv, sr, rr, sl, rl, isems):
        my = lax.axis_index("i")
        left = (my - 1) % N_DEV
        right = (my + 1) % N_DEV

        cin_a = pltpu.make_async_copy(
            x_ref.at[pl.ds(0, m_half), :], xv.at[pl.ds(0, m_half), :],
            isems.at[0])
        cin_b = pltpu.make_async_copy(
            x_ref.at[pl.ds(m_half, m_half), :], xv.at[pl.ds(m_half, m_half), :],
            isems.at[1])
        cin_a.start()
        cin_b.start()

        barrier_sem = pltpu.get_barrier_semaphore()
        for nbr in (left, right):
            pl.semaphore_signal(
                barrier_sem, inc=1,
                device_id=(nbr,), device_id_type=pl.DeviceIdType.MESH,
            )
        pl.semaphore_wait(barrier_sem, 2)

        def copy(row_start, rows, dev, ssem, rsem):
            return pltpu.make_async_remote_copy(
                src_ref=out_ref.at[pl.ds(row_start, rows), :],
                dst_ref=out_ref.at[pl.ds(row_start, rows), :],
                send_sem=ssem, recv_sem=rsem,
                device_id=(dev,), device_id_type=pl.DeviceIdType.MESH,
            )

        base = my * m_per
        r0a = copy(base, m_half, right, sr.at[0], rr.at[0])
        r0b = copy(base + m_half, m_half, right, sr.at[1], rr.at[1])
        l0a = copy(base + m_half, m_half, left, sl.at[0], rl.at[0])
        l0b = copy(base, m_half, left, sl.at[1], rl.at[1])

        cin_a.wait()
        out_ref[pl.ds(base, m_half), :] = xv[:m_half, :].astype(jnp.bfloat16)
        r0a.start()
        cin_b.wait()
        out_ref[pl.ds(base + m_half, m_half), :] = (
            xv[m_half:, :].astype(jnp.bfloat16))
        l0a.start()
        r0b.start()
        l0b.start()

        r0a.wait_recv()
        r1 = copy(left * m_per, m_half, right, sr.at[2], rr.at[2])
        r1.start()

        l0a.wait_recv()
        l1 = copy(right * m_per + m_half, m_half, left, sl.at[2], rl.at[2])
        l1.start()

        r0b.wait_recv()
        l0b.wait_recv()
        r1.wait_recv()
        l1.wait_recv()

        for rdma in (r0a, r0b, l0a, l0b, r1, l1):
            rdma.wait_send()

    x_hbm = pltpu.with_memory_space_constraint(x, pltpu.MemorySpace.HBM)
    return pl.pallas_call(
        body,
        out_shape=jax.ShapeDtypeStruct((N_DEV * m_per, n), jnp.bfloat16),
        in_specs=[
            pl.BlockSpec(memory_space=pl.ANY),
        ],
        out_specs=pl.BlockSpec(memory_space=pltpu.VMEM),
        scratch_shapes=[
            pltpu.VMEM((m_per, n), x.dtype),
            pltpu.SemaphoreType.DMA((3,)),
            pltpu.SemaphoreType.DMA((3,)),
            pltpu.SemaphoreType.DMA((3,)),
            pltpu.SemaphoreType.DMA((3,)),
            pltpu.SemaphoreType.DMA((2,)),
        ],
        compiler_params=pltpu.CompilerParams(collective_id=0),
    )(x_hbm)


# device time: 42404 ns/iter; 1.0460x vs baseline; 1.0104x over previous
import jax
import jax.numpy as jnp
from jax import lax
from jax.experimental import pallas as pl
from jax.experimental.pallas import tpu as pltpu

N_DEV = 4


def kernel(x):
    m_per, n = x.shape
    m_half = m_per // 2
    m_q = m_per // 4

    def body(x_ref, out_ref, xv, sr, rr, sl, rl, isems):
        my = lax.axis_index("i")
        left = (my - 1) % N_DEV
        right = (my + 1) % N_DEV

        def stage(qi, slot):
            c = pltpu.make_async_copy(
                x_ref.at[pl.ds(qi * m_q, m_q), :],
                xv.at[pl.ds(qi * m_q, m_q), :],
                isems.at[slot])
            c.start()
            return c

        cin = [stage(qi, slot) for slot, qi in enumerate((0, 2, 1, 3))]

        barrier_sem = pltpu.get_barrier_semaphore()
        for nbr in (left, right):
            pl.semaphore_signal(
                barrier_sem, inc=1,
                device_id=(nbr,), device_id_type=pl.DeviceIdType.MESH,
            )
        pl.semaphore_wait(barrier_sem, 2)

        def copy(row_start, rows, dev, ssem, rsem):
            return pltpu.make_async_remote_copy(
                src_ref=out_ref.at[pl.ds(row_start, rows), :],
                dst_ref=out_ref.at[pl.ds(row_start, rows), :],
                send_sem=ssem, recv_sem=rsem,
                device_id=(dev,), device_id_type=pl.DeviceIdType.MESH,
            )

        base = my * m_per

        def convert(qi):
            out_ref[pl.ds(base + qi * m_q, m_q), :] = (
                xv[pl.ds(qi * m_q, m_q), :].astype(jnp.bfloat16))

        r0 = [copy(base + qi * m_q, m_q, right, sr.at[i], rr.at[i])
              for i, qi in enumerate((0, 1, 2, 3))]
        l0 = [copy(base + qi * m_q, m_q, left, sl.at[i], rl.at[i])
              for i, qi in enumerate((2, 3, 0, 1))]

        cin[0].wait()
        convert(0)
        r0[0].start()
        cin[1].wait()
        convert(2)
        l0[0].start()
        cin[2].wait()
        convert(1)
        r0[1].start()
        l0[2].start()
        cin[3].wait()
        convert(3)
        l0[1].start()
        r0[2].start()
        r0[3].start()
        l0[3].start()

        r0[0].wait_recv()
        r0[1].wait_recv()
        r1 = copy(left * m_per, m_half, right, sr.at[4], rr.at[4])
        r1.start()

        l0[0].wait_recv()
        l0[1].wait_recv()
        l1 = copy(right * m_per + m_half, m_half, left, sl.at[4], rl.at[4])
        l1.start()

        r0[2].wait_recv()
        r0[3].wait_recv()
        l0[2].wait_recv()
        l0[3].wait_recv()
        r1.wait_recv()
        l1.wait_recv()

        for rdma in r0 + l0 + [r1, l1]:
            rdma.wait_send()

    x_hbm = pltpu.with_memory_space_constraint(x, pltpu.MemorySpace.HBM)
    return pl.pallas_call(
        body,
        out_shape=jax.ShapeDtypeStruct((N_DEV * m_per, n), jnp.bfloat16),
        in_specs=[
            pl.BlockSpec(memory_space=pl.ANY),
        ],
        out_specs=pl.BlockSpec(memory_space=pltpu.VMEM),
        scratch_shapes=[
            pltpu.VMEM((m_per, n), x.dtype),
            pltpu.SemaphoreType.DMA((5,)),
            pltpu.SemaphoreType.DMA((5,)),
            pltpu.SemaphoreType.DMA((5,)),
            pltpu.SemaphoreType.DMA((5,)),
            pltpu.SemaphoreType.DMA((4,)),
        ],
        compiler_params=pltpu.CompilerParams(collective_id=0),
    )(x_hbm)
